# baseline (speedup 1.0000x reference)
import functools

import jax
import jax.numpy as jnp
from jax import lax
from jax.experimental import pallas as pl
from jax.experimental.pallas import tpu as pltpu
from jax.experimental.pallas import tpu_sc as plsc

B = 8
N_SITES = 10000
E = 160000
BOND_LEN = 16
SITE_LEN = 128
STATE_LEN = 16
H1 = 128
H2 = 128

NC = 2
NT = 16
CH = 128
NCHUNK = E // CH
CPT_BIG, T_BIG = 80, 12
CPT_SML = 72
GRP = 8
CPW_BIG, W_BIG = 40, 28
CPW_SML = 32
RS = 624
RS_LAST = N_SITES - 15 * RS


def _sc_scatter_kernel(bonds_hbm, idx_hbm, sums_hbm, counts_hbm,
                       acc0, acc1, acc2, acc3, cacc,
                       cibuf, tibuf, bbufA, bbufB, pbuf, ones, zbuf,
                       lsemA, lsemB, ssemA, ssemB):
    c = lax.axis_index("c")
    t = lax.axis_index("s")
    w = c * NT + t
    accs = (acc0, acc1, acc2, acc3)
    start_t = jnp.where(t < T_BIG, t * CPT_BIG,
                        T_BIG * CPT_BIG + (t - T_BIG) * CPT_SML)
    start_w = jnp.where(w < W_BIG, w * CPW_BIG,
                        W_BIG * CPW_BIG + (w - W_BIG) * CPW_SML)

    def _fill(i, _):
        zbuf[i] = jnp.zeros((16,), jnp.float32)
        return 0
    lax.fori_loop(0, RS_LAST, _fill, 0)

    def _fill1(i, _):
        ones[i] = jnp.full((16,), 1.0, jnp.float32)
        return 0
    lax.fori_loop(0, CH, _fill1, 0)

    row0 = t * RS
    for a in accs + (cacc,):
        @pl.when(t < NT - 1)
        def _():
            pltpu.sync_copy(zbuf.at[pl.ds(0, RS)], a.at[pl.ds(row0, RS)])

        @pl.when(t == NT - 1)
        def _():
            pltpu.sync_copy(zbuf, a.at[pl.ds(15 * RS, RS_LAST)])

    @pl.when(t < T_BIG)
    def _():
        pltpu.sync_copy(idx_hbm.at[pl.ds(start_t, CPT_BIG)],
                        tibuf.at[pl.ds(0, CPT_BIG)])

    @pl.when(t >= T_BIG)
    def _():
        pltpu.sync_copy(idx_hbm.at[pl.ds(start_t, CPT_SML)],
                        tibuf.at[pl.ds(0, CPT_SML)])

    @pl.when(t < 2)
    def _():
        pltpu.sync_copy(idx_hbm.at[pl.ds(1248, 2)],
                        tibuf.at[pl.ds(CPT_BIG, 2)])

    @pl.when(w < W_BIG)
    def _():
        pltpu.sync_copy(idx_hbm.at[pl.ds(start_w, CPW_BIG)],
                        cibuf.at[pl.ds(0, CPW_BIG)])

    @pl.when(w >= W_BIG)
    def _():
        pltpu.sync_copy(idx_hbm.at[pl.ds(start_w, CPW_SML)],
                        cibuf.at[pl.ds(0, CPW_SML)])

    @pl.when(w < 2)
    def _():
        pltpu.sync_copy(idx_hbm.at[pl.ds(1248, 2)],
                        cibuf.at[pl.ds(CPW_BIG, 2)])

    plsc.subcore_barrier()

    for j in range(CPW_SML):
        pltpu.sync_copy(ones, cacc.at[cibuf.at[j]], add=True)

    @pl.when(w < W_BIG)
    def _():
        for j in range(CPW_SML, CPW_BIG):
            pltpu.sync_copy(ones, cacc.at[cibuf.at[j]], add=True)

    @pl.when(w == 0)
    def _():
        pltpu.sync_copy(ones, cacc.at[cibuf.at[CPW_BIG]], add=True)

    @pl.when(w == 1)
    def _():
        pltpu.sync_copy(ones, cacc.at[cibuf.at[CPW_BIG + 1]], add=True)

    NG = CPT_BIG // GRP
    slots = [(b, g) for b in range(4) for g in range(NG)]
    bbufs = (bbufA, bbufB)
    lsems = (lsemA, lsemB)
    ssems = (ssemA, ssemB)

    def _start_load(i):
        b, g = slots[i]
        valid = jnp.logical_or(g < NG - 1, t < T_BIG)
        r0 = jnp.where(valid, (start_t + g * GRP) * (CH // 8), 0)
        p = i % 2
        return [pltpu.async_copy(
            bonds_hbm.at[c * 4 + b, pl.ds(r0, GRP * 16),
                         pl.ds(cpos * 16, 16)],
            bbufs[p].at[pl.ds(cpos * CH, CH)], lsems[p])
            for cpos in range(8)]

    loads = {0: _start_load(0), 1: _start_load(1)}
    for i, (b, g) in enumerate(slots):
        p = i % 2
        valid_py = g < NG - 1
        for h in loads.pop(i):
            h.wait()
        scat = []

        def _fire():
            for cpos in range(GRP):
                scat.append(pltpu.async_copy(
                    bbufs[p].at[pl.ds(cpos * CH, CH)],
                    accs[b].at[tibuf.at[g * GRP + cpos]], ssems[p], add=True))
            for h in scat:
                h.wait()

        if valid_py:
            _fire()
        else:
            pl.when(t < T_BIG)(_fire)
        if i + 2 < len(slots):
            loads[i + 2] = _start_load(i + 2)

    def _extra(b, chunk, irow):
        for cpos in range(8):
            pltpu.sync_copy(
                bonds_hbm.at[c * 4 + b, pl.ds(chunk * 16, 16),
                             pl.ds(cpos * 16, 16)],
                pbuf.at[pl.ds(cpos * 16, 16)])
        pltpu.sync_copy(pbuf, accs[b].at[tibuf.at[irow]], add=True)

    for b in range(4):
        @pl.when(t == 0)
        def _():
            _extra(b, 1248, CPT_BIG)

        @pl.when(t == 1)
        def _():
            _extra(b, 1249, CPT_BIG + 1)

    plsc.subcore_barrier()

    outs = [(accs[b], sums_hbm, c * 4 + b) for b in range(4)]
    outs.append((cacc, counts_hbm, c))
    for src, dst, i in outs:
        @pl.when(t < NT - 1)
        def _():
            pltpu.sync_copy(src.at[pl.ds(row0, RS)],
                            dst.at[i, pl.ds(row0, RS)])

        @pl.when(t == NT - 1)
        def _():
            pltpu.sync_copy(src.at[pl.ds(15 * RS, RS_LAST)],
                            dst.at[i, pl.ds(15 * RS, RS_LAST)])


@jax.jit
def _sc_scatter(bonds, idx1d):
    mesh = plsc.VectorSubcoreMesh(core_axis_name="c", subcore_axis_name="s")
    f = functools.partial(
        pl.kernel,
        mesh=mesh,
        out_type=(
            jax.ShapeDtypeStruct((B, N_SITES, BOND_LEN), jnp.float32),
            jax.ShapeDtypeStruct((NC, N_SITES, BOND_LEN), jnp.float32),
        ),
        scratch_types=[
            pltpu.VMEM_SHARED((N_SITES, BOND_LEN), jnp.float32),
            pltpu.VMEM_SHARED((N_SITES, BOND_LEN), jnp.float32),
            pltpu.VMEM_SHARED((N_SITES, BOND_LEN), jnp.float32),
            pltpu.VMEM_SHARED((N_SITES, BOND_LEN), jnp.float32),
            pltpu.VMEM_SHARED((N_SITES, BOND_LEN), jnp.float32),
            pltpu.VMEM((CPW_BIG + 2, CH), jnp.int32),
            pltpu.VMEM((CPT_BIG + 2, CH), jnp.int32),
            pltpu.VMEM((GRP * CH, BOND_LEN), jnp.float32),
            pltpu.VMEM((GRP * CH, BOND_LEN), jnp.float32),
            pltpu.VMEM((CH, BOND_LEN), jnp.float32),
            pltpu.VMEM((CH, BOND_LEN), jnp.float32),
            pltpu.VMEM((RS_LAST, BOND_LEN), jnp.float32),
            pltpu.SemaphoreType.DMA,
            pltpu.SemaphoreType.DMA,
            pltpu.SemaphoreType.DMA,
            pltpu.SemaphoreType.DMA,
        ],
        compiler_params=pltpu.CompilerParams(use_tc_tiling_on_sc=False),
    )(_sc_scatter_kernel)
    return f(bonds, idx1d)


def _tr_kernel(src_ref, out_ref):
    x = src_ref[0]
    y = x.T.reshape(out_ref.shape[1], 8, BOND_LEN)
    for cc in range(8):
        out_ref[0, :, cc * BOND_LEN:(cc + 1) * BOND_LEN] = y[:, cc, :]


@jax.jit
def _detile(bonds_t):
    BLKE = 1280
    grid = (B, E // BLKE)
    return pl.pallas_call(
        _tr_kernel,
        grid=grid,
        in_specs=[pl.BlockSpec((1, BOND_LEN, BLKE), lambda b, j: (b, 0, j))],
        out_specs=pl.BlockSpec((1, BLKE // 8, CH), lambda b, j: (b, j, 0)),
        out_shape=jax.ShapeDtypeStruct((B, E // 8, CH), jnp.float32),
        compiler_params=pltpu.CompilerParams(
            dimension_semantics=("parallel", "parallel"),
        ),
    )(bonds_t)


def _mlp_kernel(sums_ref, counts_ref, sites_ref, states_ref,
                w1p_ref, w1s_ref, w1t_ref, b1_ref, w2_ref, b2_ref,
                w3_ref, b3_ref, out_ref):
    cnt = counts_ref[0] + counts_ref[1]
    pool = sums_ref[0] / jnp.maximum(cnt, 1.0)
    st = states_ref[pl.ds(pl.program_id(0), 1)]
    bias = (jnp.dot(st, w1t_ref[...],
                    preferred_element_type=jnp.float32)
            + b1_ref[...])
    h = (jnp.dot(pool, w1p_ref[...],
                 preferred_element_type=jnp.float32)
         + jnp.dot(sites_ref[0], w1s_ref[...],
                   preferred_element_type=jnp.float32)
         + bias)
    h = jnp.maximum(h, 0.0)
    h = jnp.maximum(jnp.dot(h, w2_ref[...],
                            preferred_element_type=jnp.float32)
                    + b2_ref[...], 0.0)
    out_ref[0] = jnp.maximum(jnp.dot(h, w3_ref[...],
                                     preferred_element_type=jnp.float32)
                             + b3_ref[...], 0.0)


@jax.jit
def _mlp(sums, counts, sites, states, w1p, w1s, w1t, b1, w2, b2, w3, b3):
    BLK = 1000
    grid = (B, N_SITES // BLK)
    return pl.pallas_call(
        _mlp_kernel,
        grid=grid,
        in_specs=[
            pl.BlockSpec((1, BLK, BOND_LEN), lambda b, j: (b, j, 0)),
            pl.BlockSpec((NC, BLK, BOND_LEN), lambda b, j: (0, j, 0)),
            pl.BlockSpec((1, BLK, SITE_LEN), lambda b, j: (b, j, 0)),
            pl.BlockSpec((B, STATE_LEN), lambda b, j: (0, 0)),
            pl.BlockSpec((BOND_LEN, H1), lambda b, j: (0, 0)),
            pl.BlockSpec((SITE_LEN, H1), lambda b, j: (0, 0)),
            pl.BlockSpec((STATE_LEN, H1), lambda b, j: (0, 0)),
            pl.BlockSpec((1, H1), lambda b, j: (0, 0)),
            pl.BlockSpec((H1, H2), lambda b, j: (0, 0)),
            pl.BlockSpec((1, H2), lambda b, j: (0, 0)),
            pl.BlockSpec((H2, SITE_LEN), lambda b, j: (0, 0)),
            pl.BlockSpec((1, SITE_LEN), lambda b, j: (0, 0)),
        ],
        out_specs=pl.BlockSpec((1, BLK, SITE_LEN), lambda b, j: (b, j, 0)),
        out_shape=jax.ShapeDtypeStruct((B, N_SITES, SITE_LEN), jnp.float32),
        compiler_params=pltpu.CompilerParams(
            dimension_semantics=("parallel", "parallel"),
        ),
    )(sums, counts, sites, states, w1p, w1s, w1t, b1, w2, b2, w3, b3)


def kernel(sites, bonds, states, indices1, W1, b1, W2, b2, W3, b3):
    ngrp = (NCHUNK // GRP) * GRP * CH
    idx_perm = jnp.concatenate([
        indices1[:ngrp].reshape(-1, CH, 8).transpose(0, 2, 1).reshape(-1, CH),
        indices1[ngrp:].reshape(2, 16, 8).transpose(0, 2, 1).reshape(2, CH),
    ], axis=0)
    sums, counts = _sc_scatter(_detile(bonds.transpose(0, 2, 1)), idx_perm)
    w1p = W1[:BOND_LEN]
    w1s = W1[BOND_LEN:BOND_LEN + SITE_LEN]
    w1t = W1[BOND_LEN + SITE_LEN:]
    return _mlp(sums, counts, sites, states, w1p, w1s, w1t,
                b1.reshape(1, H1), W2, b2.reshape(1, H2),
                W3, b3.reshape(1, SITE_LEN))


# speedup vs baseline: 1.3809x; 1.3809x over previous
import functools

import jax
import jax.numpy as jnp
from jax import lax
from jax.experimental import pallas as pl
from jax.experimental.pallas import tpu as pltpu
from jax.experimental.pallas import tpu_sc as plsc

B = 8
N_SITES = 10000
E = 160000
BOND_LEN = 16
SITE_LEN = 128
STATE_LEN = 16
H1 = 128
H2 = 128

NC = 2
NT = 16
CH = 128
NCHUNK = E // CH
CPT_BIG, T_BIG = 80, 12
CPT_SML = 72
GRP = 8
CPW_BIG, W_BIG = 40, 28
CPW_SML = 32
RS = 624
RS_LAST = N_SITES - 15 * RS


def _sc_scatter_kernel(bonds_hbm, idx_hbm, sums_hbm, counts_hbm,
                       acc0, acc1, acc2, acc3, cacc,
                       cibuf, tibuf, bbufA, bbufB, pbuf, ones, zbuf,
                       lsemA, lsemB, ssemA, ssemB):
    c = lax.axis_index("c")
    t = lax.axis_index("s")
    w = c * NT + t
    accs = (acc0, acc1, acc2, acc3)
    start_t = jnp.where(t < T_BIG, t * CPT_BIG,
                        T_BIG * CPT_BIG + (t - T_BIG) * CPT_SML)
    start_w = jnp.where(w < W_BIG, w * CPW_BIG,
                        W_BIG * CPW_BIG + (w - W_BIG) * CPW_SML)

    def _fill(i, _):
        zbuf[i] = jnp.zeros((16,), jnp.float32)
        return 0
    lax.fori_loop(0, RS_LAST, _fill, 0)

    def _fill1(i, _):
        ones[i] = jnp.full((16,), 1.0, jnp.float32)
        return 0
    lax.fori_loop(0, CH, _fill1, 0)

    row0 = t * RS
    for a in accs + (cacc,):
        @pl.when(t < NT - 1)
        def _():
            pltpu.sync_copy(zbuf.at[pl.ds(0, RS)], a.at[pl.ds(row0, RS)])

        @pl.when(t == NT - 1)
        def _():
            pltpu.sync_copy(zbuf, a.at[pl.ds(15 * RS, RS_LAST)])

    @pl.when(t < T_BIG)
    def _():
        pltpu.sync_copy(idx_hbm.at[pl.ds(start_t, CPT_BIG)],
                        tibuf.at[pl.ds(0, CPT_BIG)])

    @pl.when(t >= T_BIG)
    def _():
        pltpu.sync_copy(idx_hbm.at[pl.ds(start_t, CPT_SML)],
                        tibuf.at[pl.ds(0, CPT_SML)])

    @pl.when(t < 2)
    def _():
        pltpu.sync_copy(idx_hbm.at[pl.ds(1248, 2)],
                        tibuf.at[pl.ds(CPT_BIG, 2)])

    @pl.when(w < W_BIG)
    def _():
        pltpu.sync_copy(idx_hbm.at[pl.ds(start_w, CPW_BIG)],
                        cibuf.at[pl.ds(0, CPW_BIG)])

    @pl.when(w >= W_BIG)
    def _():
        pltpu.sync_copy(idx_hbm.at[pl.ds(start_w, CPW_SML)],
                        cibuf.at[pl.ds(0, CPW_SML)])

    @pl.when(w < 2)
    def _():
        pltpu.sync_copy(idx_hbm.at[pl.ds(1248, 2)],
                        cibuf.at[pl.ds(CPW_BIG, 2)])

    plsc.subcore_barrier()

    for j in range(CPW_SML):
        pltpu.sync_copy(ones, cacc.at[cibuf.at[j]], add=True)

    @pl.when(w < W_BIG)
    def _():
        for j in range(CPW_SML, CPW_BIG):
            pltpu.sync_copy(ones, cacc.at[cibuf.at[j]], add=True)

    @pl.when(w == 0)
    def _():
        pltpu.sync_copy(ones, cacc.at[cibuf.at[CPW_BIG]], add=True)

    @pl.when(w == 1)
    def _():
        pltpu.sync_copy(ones, cacc.at[cibuf.at[CPW_BIG + 1]], add=True)

    NG = CPT_BIG // GRP
    slots = [(b, g) for b in range(4) for g in range(NG)]
    bbufs = (bbufA, bbufB)
    lsems = (lsemA, lsemB)
    ssems = (ssemA, ssemB)

    def _start_load(i):
        b, g = slots[i]
        valid = jnp.logical_or(g < NG - 1, t < T_BIG)
        e0 = jnp.where(valid, (start_t + g * GRP) * CH, 0)
        p = i % 2
        return pltpu.async_copy(bonds_hbm.at[c * 4 + b, pl.ds(e0, GRP * CH)],
                                bbufs[p], lsems[p])

    loads = {0: _start_load(0), 1: _start_load(1)}
    for i, (b, g) in enumerate(slots):
        p = i % 2
        valid_py = g < NG - 1
        loads.pop(i).wait()
        scat = []

        def _fire():
            for cpos in range(GRP):
                scat.append(pltpu.async_copy(
                    bbufs[p].at[pl.ds(cpos * CH, CH)],
                    accs[b].at[tibuf.at[g * GRP + cpos]], ssems[p], add=True))
            for h in scat:
                h.wait()

        if valid_py:
            _fire()
        else:
            pl.when(t < T_BIG)(_fire)
        if i + 2 < len(slots):
            loads[i + 2] = _start_load(i + 2)

    def _extra(b, chunk, irow):
        pltpu.sync_copy(bonds_hbm.at[c * 4 + b, pl.ds(chunk * CH, CH)], pbuf)
        pltpu.sync_copy(pbuf, accs[b].at[tibuf.at[irow]], add=True)

    for b in range(4):
        @pl.when(t == 0)
        def _():
            _extra(b, 1248, CPT_BIG)

        @pl.when(t == 1)
        def _():
            _extra(b, 1249, CPT_BIG + 1)

    plsc.subcore_barrier()

    outs = [(accs[b], sums_hbm, c * 4 + b) for b in range(4)]
    outs.append((cacc, counts_hbm, c))
    for src, dst, i in outs:
        @pl.when(t < NT - 1)
        def _():
            pltpu.sync_copy(src.at[pl.ds(row0, RS)],
                            dst.at[i, pl.ds(row0, RS)])

        @pl.when(t == NT - 1)
        def _():
            pltpu.sync_copy(src.at[pl.ds(15 * RS, RS_LAST)],
                            dst.at[i, pl.ds(15 * RS, RS_LAST)])


@jax.jit
def _sc_scatter(bonds, idx1d):
    mesh = plsc.VectorSubcoreMesh(core_axis_name="c", subcore_axis_name="s")
    f = functools.partial(
        pl.kernel,
        mesh=mesh,
        out_type=(
            jax.ShapeDtypeStruct((B, N_SITES, BOND_LEN), jnp.float32),
            jax.ShapeDtypeStruct((NC, N_SITES, BOND_LEN), jnp.float32),
        ),
        scratch_types=[
            pltpu.VMEM_SHARED((N_SITES, BOND_LEN), jnp.float32),
            pltpu.VMEM_SHARED((N_SITES, BOND_LEN), jnp.float32),
            pltpu.VMEM_SHARED((N_SITES, BOND_LEN), jnp.float32),
            pltpu.VMEM_SHARED((N_SITES, BOND_LEN), jnp.float32),
            pltpu.VMEM_SHARED((N_SITES, BOND_LEN), jnp.float32),
            pltpu.VMEM((CPW_BIG + 2, CH), jnp.int32),
            pltpu.VMEM((CPT_BIG + 2, CH), jnp.int32),
            pltpu.VMEM((GRP * CH, BOND_LEN), jnp.float32),
            pltpu.VMEM((GRP * CH, BOND_LEN), jnp.float32),
            pltpu.VMEM((CH, BOND_LEN), jnp.float32),
            pltpu.VMEM((CH, BOND_LEN), jnp.float32),
            pltpu.VMEM((RS_LAST, BOND_LEN), jnp.float32),
            pltpu.SemaphoreType.DMA,
            pltpu.SemaphoreType.DMA,
            pltpu.SemaphoreType.DMA,
            pltpu.SemaphoreType.DMA,
        ],
        compiler_params=pltpu.CompilerParams(use_tc_tiling_on_sc=False),
    )(_sc_scatter_kernel)
    return f(bonds, idx1d)


def _mlp_kernel(sums_ref, counts_ref, sites_ref, states_ref,
                w1p_ref, w1s_ref, w1t_ref, b1_ref, w2_ref, b2_ref,
                w3_ref, b3_ref, out_ref):
    cnt = counts_ref[0] + counts_ref[1]
    pool = sums_ref[0] / jnp.maximum(cnt, 1.0)
    st = states_ref[pl.ds(pl.program_id(0), 1)]
    bias = (jnp.dot(st, w1t_ref[...],
                    preferred_element_type=jnp.float32)
            + b1_ref[...])
    h = (jnp.dot(pool, w1p_ref[...],
                 preferred_element_type=jnp.float32)
         + jnp.dot(sites_ref[0], w1s_ref[...],
                   preferred_element_type=jnp.float32)
         + bias)
    h = jnp.maximum(h, 0.0)
    h = jnp.maximum(jnp.dot(h, w2_ref[...],
                            preferred_element_type=jnp.float32)
                    + b2_ref[...], 0.0)
    out_ref[0] = jnp.maximum(jnp.dot(h, w3_ref[...],
                                     preferred_element_type=jnp.float32)
                             + b3_ref[...], 0.0)


@jax.jit
def _mlp(sums, counts, sites, states, w1p, w1s, w1t, b1, w2, b2, w3, b3):
    BLK = 1000
    grid = (B, N_SITES // BLK)
    return pl.pallas_call(
        _mlp_kernel,
        grid=grid,
        in_specs=[
            pl.BlockSpec((1, BLK, BOND_LEN), lambda b, j: (b, j, 0)),
            pl.BlockSpec((NC, BLK, BOND_LEN), lambda b, j: (0, j, 0)),
            pl.BlockSpec((1, BLK, SITE_LEN), lambda b, j: (b, j, 0)),
            pl.BlockSpec((B, STATE_LEN), lambda b, j: (0, 0)),
            pl.BlockSpec((BOND_LEN, H1), lambda b, j: (0, 0)),
            pl.BlockSpec((SITE_LEN, H1), lambda b, j: (0, 0)),
            pl.BlockSpec((STATE_LEN, H1), lambda b, j: (0, 0)),
            pl.BlockSpec((1, H1), lambda b, j: (0, 0)),
            pl.BlockSpec((H1, H2), lambda b, j: (0, 0)),
            pl.BlockSpec((1, H2), lambda b, j: (0, 0)),
            pl.BlockSpec((H2, SITE_LEN), lambda b, j: (0, 0)),
            pl.BlockSpec((1, SITE_LEN), lambda b, j: (0, 0)),
        ],
        out_specs=pl.BlockSpec((1, BLK, SITE_LEN), lambda b, j: (b, j, 0)),
        out_shape=jax.ShapeDtypeStruct((B, N_SITES, SITE_LEN), jnp.float32),
        compiler_params=pltpu.CompilerParams(
            dimension_semantics=("parallel", "parallel"),
        ),
    )(sums, counts, sites, states, w1p, w1s, w1t, b1, w2, b2, w3, b3)


def kernel(sites, bonds, states, indices1, W1, b1, W2, b2, W3, b3):
    sums, counts = _sc_scatter(bonds, indices1.reshape(NCHUNK, CH))
    w1p = W1[:BOND_LEN]
    w1s = W1[BOND_LEN:BOND_LEN + SITE_LEN]
    w1t = W1[BOND_LEN + SITE_LEN:]
    return _mlp(sums, counts, sites, states, w1p, w1s, w1t,
                b1.reshape(1, H1), W2, b2.reshape(1, H2),
                W3, b3.reshape(1, SITE_LEN))


# speedup vs baseline: 1.4457x; 1.0469x over previous
import functools

import jax
import jax.numpy as jnp
from jax import lax
from jax.experimental import pallas as pl
from jax.experimental.pallas import tpu as pltpu
from jax.experimental.pallas import tpu_sc as plsc

B = 8
N_SITES = 10000
E = 160000
BOND_LEN = 16
SITE_LEN = 128
STATE_LEN = 16
H1 = 128
H2 = 128

NC = 2
NT = 16
CH = 128
NCHUNK = E // CH
CPT_BIG, T_BIG = 80, 12
CPT_SML = 72
GRP = 8
CPW_BIG, W_BIG = 40, 28
CPW_SML = 32
RS = 624
RS_LAST = N_SITES - 15 * RS


def _sc_scatter_kernel(bonds_hbm, idx_hbm, sums_hbm, counts_hbm,
                       acc0, acc1, acc2, acc3, cacc,
                       cibuf, tibuf, bbufA, bbufB, pbuf, ones, zbuf,
                       lsemA, lsemB, ssemA, ssemB):
    c = lax.axis_index("c")
    t = lax.axis_index("s")
    w = c * NT + t
    accs = (acc0, acc1, acc2, acc3)
    start_t = jnp.where(t < T_BIG, t * CPT_BIG,
                        T_BIG * CPT_BIG + (t - T_BIG) * CPT_SML)
    start_w = jnp.where(w < W_BIG, w * CPW_BIG,
                        W_BIG * CPW_BIG + (w - W_BIG) * CPW_SML)

    def _fill(i, _):
        zbuf[i] = jnp.zeros((16,), jnp.float32)
        return 0
    lax.fori_loop(0, RS_LAST, _fill, 0)

    def _fill1(i, _):
        ones[i] = jnp.full((16,), 1.0, jnp.float32)
        return 0
    lax.fori_loop(0, CH, _fill1, 0)

    row0 = t * RS
    for a in accs + (cacc,):
        @pl.when(t < NT - 1)
        def _():
            pltpu.sync_copy(zbuf.at[pl.ds(0, RS)], a.at[pl.ds(row0, RS)])

        @pl.when(t == NT - 1)
        def _():
            pltpu.sync_copy(zbuf, a.at[pl.ds(15 * RS, RS_LAST)])

    @pl.when(t < T_BIG)
    def _():
        pltpu.sync_copy(idx_hbm.at[pl.ds(start_t, CPT_BIG)],
                        tibuf.at[pl.ds(0, CPT_BIG)])

    @pl.when(t >= T_BIG)
    def _():
        pltpu.sync_copy(idx_hbm.at[pl.ds(start_t, CPT_SML)],
                        tibuf.at[pl.ds(0, CPT_SML)])

    @pl.when(t < 2)
    def _():
        pltpu.sync_copy(idx_hbm.at[pl.ds(1248, 2)],
                        tibuf.at[pl.ds(CPT_BIG, 2)])

    @pl.when(w < W_BIG)
    def _():
        pltpu.sync_copy(idx_hbm.at[pl.ds(start_w, CPW_BIG)],
                        cibuf.at[pl.ds(0, CPW_BIG)])

    @pl.when(w >= W_BIG)
    def _():
        pltpu.sync_copy(idx_hbm.at[pl.ds(start_w, CPW_SML)],
                        cibuf.at[pl.ds(0, CPW_SML)])

    @pl.when(w < 2)
    def _():
        pltpu.sync_copy(idx_hbm.at[pl.ds(1248, 2)],
                        cibuf.at[pl.ds(CPW_BIG, 2)])

    plsc.subcore_barrier()

    for j in range(CPW_SML):
        pltpu.sync_copy(ones, cacc.at[cibuf.at[j]], add=True)

    @pl.when(w < W_BIG)
    def _():
        for j in range(CPW_SML, CPW_BIG):
            pltpu.sync_copy(ones, cacc.at[cibuf.at[j]], add=True)

    @pl.when(w == 0)
    def _():
        pltpu.sync_copy(ones, cacc.at[cibuf.at[CPW_BIG]], add=True)

    @pl.when(w == 1)
    def _():
        pltpu.sync_copy(ones, cacc.at[cibuf.at[CPW_BIG + 1]], add=True)

    NG = CPT_BIG // GRP
    slots = [(b, g) for b in range(4) for g in range(NG)]
    bbufs = (bbufA, bbufB)
    lsems = (lsemA, lsemB)
    ssems = (ssemA, ssemB)

    def _start_load(i):
        b, g = slots[i]
        valid = jnp.logical_or(g < NG - 1, t < T_BIG)
        e0 = jnp.where(valid, (start_t + g * GRP) * CH, 0)
        p = i % 2
        return pltpu.async_copy(bonds_hbm.at[c * 4 + b, pl.ds(e0, GRP * CH)],
                                bbufs[p], lsems[p])

    loads = {0: _start_load(0), 1: _start_load(1)}
    for i, (b, g) in enumerate(slots):
        p = i % 2
        valid_py = g < NG - 1
        loads.pop(i).wait()
        scat = []

        def _fire():
            for cpos in range(GRP):
                scat.append(pltpu.async_copy(
                    bbufs[p].at[pl.ds(cpos * CH, CH)],
                    accs[b].at[tibuf.at[g * GRP + cpos]], ssems[p], add=True))
            for h in scat:
                h.wait()

        if valid_py:
            _fire()
        else:
            pl.when(t < T_BIG)(_fire)
        if i + 2 < len(slots):
            loads[i + 2] = _start_load(i + 2)

    def _extra(b, chunk, irow):
        pltpu.sync_copy(bonds_hbm.at[c * 4 + b, pl.ds(chunk * CH, CH)], pbuf)
        pltpu.sync_copy(pbuf, accs[b].at[tibuf.at[irow]], add=True)

    for b in range(4):
        @pl.when(t == 0)
        def _():
            _extra(b, 1248, CPT_BIG)

        @pl.when(t == 1)
        def _():
            _extra(b, 1249, CPT_BIG + 1)

    plsc.subcore_barrier()

    outs = [(accs[b], sums_hbm, c * 4 + b) for b in range(4)]
    outs.append((cacc, counts_hbm, c))
    for src, dst, i in outs:
        @pl.when(t < NT - 1)
        def _():
            pltpu.sync_copy(src.at[pl.ds(row0, RS)],
                            dst.at[i, pl.ds(row0, RS)])

        @pl.when(t == NT - 1)
        def _():
            pltpu.sync_copy(src.at[pl.ds(15 * RS, RS_LAST)],
                            dst.at[i, pl.ds(15 * RS, RS_LAST)])


@jax.jit
def _sc_scatter(bonds, idx1d):
    mesh = plsc.VectorSubcoreMesh(core_axis_name="c", subcore_axis_name="s")
    f = functools.partial(
        pl.kernel,
        mesh=mesh,
        out_type=(
            jax.ShapeDtypeStruct((B, N_SITES, BOND_LEN), jnp.float32),
            jax.ShapeDtypeStruct((NC, N_SITES, BOND_LEN), jnp.float32),
        ),
        scratch_types=[
            pltpu.VMEM_SHARED((N_SITES, BOND_LEN), jnp.float32),
            pltpu.VMEM_SHARED((N_SITES, BOND_LEN), jnp.float32),
            pltpu.VMEM_SHARED((N_SITES, BOND_LEN), jnp.float32),
            pltpu.VMEM_SHARED((N_SITES, BOND_LEN), jnp.float32),
            pltpu.VMEM_SHARED((N_SITES, BOND_LEN), jnp.float32),
            pltpu.VMEM((CPW_BIG + 2, CH), jnp.int32),
            pltpu.VMEM((CPT_BIG + 2, CH), jnp.int32),
            pltpu.VMEM((GRP * CH, BOND_LEN), jnp.float32),
            pltpu.VMEM((GRP * CH, BOND_LEN), jnp.float32),
            pltpu.VMEM((CH, BOND_LEN), jnp.float32),
            pltpu.VMEM((CH, BOND_LEN), jnp.float32),
            pltpu.VMEM((RS_LAST, BOND_LEN), jnp.float32),
            pltpu.SemaphoreType.DMA,
            pltpu.SemaphoreType.DMA,
            pltpu.SemaphoreType.DMA,
            pltpu.SemaphoreType.DMA,
        ],
        compiler_params=pltpu.CompilerParams(use_tc_tiling_on_sc=False),
    )(_sc_scatter_kernel)
    return f(bonds, idx1d)


def _mlp_kernel(sums_ref, counts_ref, sites_ref, states_ref,
                w1p_ref, w1s_ref, w1t_ref, b1_ref, w2_ref, b2_ref,
                w3_ref, b3_ref, out_ref):
    cnt = counts_ref[0] + counts_ref[1]
    pool = sums_ref[0] / jnp.maximum(cnt, 1.0)
    st = states_ref[pl.ds(pl.program_id(0), 1)]
    bias = (jnp.dot(st, w1t_ref[...],
                    preferred_element_type=jnp.float32)
            + b1_ref[...])
    h = (jnp.dot(pool, w1p_ref[...],
                 preferred_element_type=jnp.float32)
         + jnp.dot(sites_ref[0], w1s_ref[...],
                   preferred_element_type=jnp.float32)
         + bias)
    h = jnp.maximum(h, 0.0)
    h = jnp.maximum(jnp.dot(h, w2_ref[...],
                            preferred_element_type=jnp.float32)
                    + b2_ref[...], 0.0)
    out_ref[0] = jnp.maximum(jnp.dot(h, w3_ref[...],
                                     preferred_element_type=jnp.float32)
                             + b3_ref[...], 0.0)


@jax.jit
def _mlp(sums, counts, sites, states, w1p, w1s, w1t, b1, w2, b2, w3, b3):
    BLK = 2000
    grid = (B, N_SITES // BLK)
    return pl.pallas_call(
        _mlp_kernel,
        grid=grid,
        in_specs=[
            pl.BlockSpec((1, BLK, BOND_LEN), lambda b, j: (b, j, 0)),
            pl.BlockSpec((NC, BLK, BOND_LEN), lambda b, j: (0, j, 0)),
            pl.BlockSpec((1, BLK, SITE_LEN), lambda b, j: (b, j, 0)),
            pl.BlockSpec((B, STATE_LEN), lambda b, j: (0, 0)),
            pl.BlockSpec((BOND_LEN, H1), lambda b, j: (0, 0)),
            pl.BlockSpec((SITE_LEN, H1), lambda b, j: (0, 0)),
            pl.BlockSpec((STATE_LEN, H1), lambda b, j: (0, 0)),
            pl.BlockSpec((1, H1), lambda b, j: (0, 0)),
            pl.BlockSpec((H1, H2), lambda b, j: (0, 0)),
            pl.BlockSpec((1, H2), lambda b, j: (0, 0)),
            pl.BlockSpec((H2, SITE_LEN), lambda b, j: (0, 0)),
            pl.BlockSpec((1, SITE_LEN), lambda b, j: (0, 0)),
        ],
        out_specs=pl.BlockSpec((1, BLK, SITE_LEN), lambda b, j: (b, j, 0)),
        out_shape=jax.ShapeDtypeStruct((B, N_SITES, SITE_LEN), jnp.float32),
        compiler_params=pltpu.CompilerParams(
            dimension_semantics=("parallel", "parallel"),
        ),
    )(sums, counts, sites, states, w1p, w1s, w1t, b1, w2, b2, w3, b3)


def kernel(sites, bonds, states, indices1, W1, b1, W2, b2, W3, b3):
    sums, counts = _sc_scatter(bonds, indices1.reshape(NCHUNK, CH))
    w1p = W1[:BOND_LEN]
    w1s = W1[BOND_LEN:BOND_LEN + SITE_LEN]
    w1t = W1[BOND_LEN + SITE_LEN:]
    return _mlp(sums, counts, sites, states, w1p, w1s, w1t,
                b1.reshape(1, H1), W2, b2.reshape(1, H2),
                W3, b3.reshape(1, SITE_LEN))


# speedup vs baseline: 1.4647x; 1.0132x over previous
import functools

import jax
import jax.numpy as jnp
from jax import lax
from jax.experimental import pallas as pl
from jax.experimental.pallas import tpu as pltpu
from jax.experimental.pallas import tpu_sc as plsc

B = 8
N_SITES = 10000
E = 160000
BOND_LEN = 16
SITE_LEN = 128
STATE_LEN = 16
H1 = 128
H2 = 128

NC = 2
NT = 16
CH = 128
NCHUNK = E // CH
CPT_BIG, T_BIG = 80, 12
CPT_SML = 72
GRP = 8
CPW_BIG, W_BIG = 40, 28
CPW_SML = 32
RS = 624
RS_LAST = N_SITES - 15 * RS


def _sc_scatter_kernel(bonds_hbm, idx_hbm, sums_hbm, counts_hbm,
                       acc0, acc1, acc2, acc3, cacc,
                       cibuf, tibuf, bbufA, bbufB, pbuf, ones, zbuf,
                       lsemA, lsemB, ssemA, ssemB):
    c = lax.axis_index("c")
    t = lax.axis_index("s")
    w = c * NT + t
    accs = (acc0, acc1, acc2, acc3)
    start_t = jnp.where(t < T_BIG, t * CPT_BIG,
                        T_BIG * CPT_BIG + (t - T_BIG) * CPT_SML)
    start_w = jnp.where(w < W_BIG, w * CPW_BIG,
                        W_BIG * CPW_BIG + (w - W_BIG) * CPW_SML)

    def _fill(i, _):
        zbuf[i] = jnp.zeros((16,), jnp.float32)
        return 0
    lax.fori_loop(0, RS_LAST, _fill, 0)

    def _fill1(i, _):
        ones[i] = jnp.full((16,), 1.0, jnp.float32)
        return 0
    lax.fori_loop(0, CH, _fill1, 0)

    row0 = t * RS
    for a in accs + (cacc,):
        @pl.when(t < NT - 1)
        def _():
            pltpu.sync_copy(zbuf.at[pl.ds(0, RS)], a.at[pl.ds(row0, RS)])

        @pl.when(t == NT - 1)
        def _():
            pltpu.sync_copy(zbuf, a.at[pl.ds(15 * RS, RS_LAST)])

    @pl.when(t < T_BIG)
    def _():
        pltpu.sync_copy(idx_hbm.at[pl.ds(start_t, CPT_BIG)],
                        tibuf.at[pl.ds(0, CPT_BIG)])

    @pl.when(t >= T_BIG)
    def _():
        pltpu.sync_copy(idx_hbm.at[pl.ds(start_t, CPT_SML)],
                        tibuf.at[pl.ds(0, CPT_SML)])

    @pl.when(t < 2)
    def _():
        pltpu.sync_copy(idx_hbm.at[pl.ds(1248, 2)],
                        tibuf.at[pl.ds(CPT_BIG, 2)])

    @pl.when(w < W_BIG)
    def _():
        pltpu.sync_copy(idx_hbm.at[pl.ds(start_w, CPW_BIG)],
                        cibuf.at[pl.ds(0, CPW_BIG)])

    @pl.when(w >= W_BIG)
    def _():
        pltpu.sync_copy(idx_hbm.at[pl.ds(start_w, CPW_SML)],
                        cibuf.at[pl.ds(0, CPW_SML)])

    @pl.when(w < 2)
    def _():
        pltpu.sync_copy(idx_hbm.at[pl.ds(1248, 2)],
                        cibuf.at[pl.ds(CPW_BIG, 2)])

    plsc.subcore_barrier()

    for j in range(CPW_SML):
        pltpu.sync_copy(ones, cacc.at[cibuf.at[j]], add=True)

    @pl.when(w < W_BIG)
    def _():
        for j in range(CPW_SML, CPW_BIG):
            pltpu.sync_copy(ones, cacc.at[cibuf.at[j]], add=True)

    @pl.when(w == 0)
    def _():
        pltpu.sync_copy(ones, cacc.at[cibuf.at[CPW_BIG]], add=True)

    @pl.when(w == 1)
    def _():
        pltpu.sync_copy(ones, cacc.at[cibuf.at[CPW_BIG + 1]], add=True)

    NG = CPT_BIG // GRP
    slots = [(b, g) for b in range(4) for g in range(NG)]
    bbufs = (bbufA, bbufB)
    lsems = (lsemA, lsemB)
    ssems = (ssemA, ssemB)

    def _start_load(i):
        b, g = slots[i]
        valid = jnp.logical_or(g < NG - 1, t < T_BIG)
        e0 = jnp.where(valid, (start_t + g * GRP) * CH, 0)
        p = i % 2
        return pltpu.async_copy(bonds_hbm.at[c * 4 + b, pl.ds(e0, GRP * CH)],
                                bbufs[p], lsems[p])

    loads = {0: _start_load(0), 1: _start_load(1)}
    for i, (b, g) in enumerate(slots):
        p = i % 2
        valid_py = g < NG - 1
        loads.pop(i).wait()
        scat = []

        def _fire():
            for cpos in range(GRP):
                scat.append(pltpu.async_copy(
                    bbufs[p].at[pl.ds(cpos * CH, CH)],
                    accs[b].at[tibuf.at[g * GRP + cpos]], ssems[p], add=True))
            for h in scat:
                h.wait()

        if valid_py:
            _fire()
        else:
            pl.when(t < T_BIG)(_fire)
        if i + 2 < len(slots):
            loads[i + 2] = _start_load(i + 2)

    def _extra(b, chunk, irow):
        pltpu.sync_copy(bonds_hbm.at[c * 4 + b, pl.ds(chunk * CH, CH)], pbuf)
        pltpu.sync_copy(pbuf, accs[b].at[tibuf.at[irow]], add=True)

    for b in range(4):
        @pl.when(t == 0)
        def _():
            _extra(b, 1248, CPT_BIG)

        @pl.when(t == 1)
        def _():
            _extra(b, 1249, CPT_BIG + 1)

    plsc.subcore_barrier()

    outs = [(accs[b], sums_hbm, c * 4 + b) for b in range(4)]
    outs.append((cacc, counts_hbm, c))
    for src, dst, i in outs:
        @pl.when(t < NT - 1)
        def _():
            pltpu.sync_copy(src.at[pl.ds(row0, RS)],
                            dst.at[i, pl.ds(row0, RS)])

        @pl.when(t == NT - 1)
        def _():
            pltpu.sync_copy(src.at[pl.ds(15 * RS, RS_LAST)],
                            dst.at[i, pl.ds(15 * RS, RS_LAST)])


@jax.jit
def _sc_scatter(bonds, idx1d):
    mesh = plsc.VectorSubcoreMesh(core_axis_name="c", subcore_axis_name="s")
    f = functools.partial(
        pl.kernel,
        mesh=mesh,
        out_type=(
            jax.ShapeDtypeStruct((B, N_SITES, BOND_LEN), jnp.float32),
            jax.ShapeDtypeStruct((NC, N_SITES, BOND_LEN), jnp.float32),
        ),
        scratch_types=[
            pltpu.VMEM_SHARED((N_SITES, BOND_LEN), jnp.float32),
            pltpu.VMEM_SHARED((N_SITES, BOND_LEN), jnp.float32),
            pltpu.VMEM_SHARED((N_SITES, BOND_LEN), jnp.float32),
            pltpu.VMEM_SHARED((N_SITES, BOND_LEN), jnp.float32),
            pltpu.VMEM_SHARED((N_SITES, BOND_LEN), jnp.float32),
            pltpu.VMEM((CPW_BIG + 2, CH), jnp.int32),
            pltpu.VMEM((CPT_BIG + 2, CH), jnp.int32),
            pltpu.VMEM((GRP * CH, BOND_LEN), jnp.float32),
            pltpu.VMEM((GRP * CH, BOND_LEN), jnp.float32),
            pltpu.VMEM((CH, BOND_LEN), jnp.float32),
            pltpu.VMEM((CH, BOND_LEN), jnp.float32),
            pltpu.VMEM((RS_LAST, BOND_LEN), jnp.float32),
            pltpu.SemaphoreType.DMA,
            pltpu.SemaphoreType.DMA,
            pltpu.SemaphoreType.DMA,
            pltpu.SemaphoreType.DMA,
        ],
        compiler_params=pltpu.CompilerParams(use_tc_tiling_on_sc=False),
    )(_sc_scatter_kernel)
    return f(bonds, idx1d)


def _mlp_kernel(sums_ref, counts_ref, sites_ref, states_ref,
                w1p_ref, w1s_ref, w1t_ref, b1_ref, w2_ref, b2_ref,
                w3_ref, b3_ref, out_ref):
    cnt = counts_ref[0] + counts_ref[1]
    pool = sums_ref[0] / jnp.maximum(cnt, 1.0)
    st = states_ref[pl.ds(pl.program_id(0), 1)]
    bias = (jnp.dot(st, w1t_ref[...],
                    preferred_element_type=jnp.float32)
            + b1_ref[...])
    h = (jnp.dot(pool, w1p_ref[...],
                 preferred_element_type=jnp.float32)
         + jnp.dot(sites_ref[0], w1s_ref[...],
                   preferred_element_type=jnp.float32)
         + bias)
    h = jnp.maximum(h, 0.0)
    h = jnp.maximum(jnp.dot(h, w2_ref[...],
                            preferred_element_type=jnp.float32)
                    + b2_ref[...], 0.0)
    out_ref[0] = jnp.maximum(jnp.dot(h, w3_ref[...],
                                     preferred_element_type=jnp.float32)
                             + b3_ref[...], 0.0)


@jax.jit
def _mlp(sums, counts, sites, states, w1p, w1s, w1t, b1, w2, b2, w3, b3):
    BLK = 5000
    grid = (B, N_SITES // BLK)
    return pl.pallas_call(
        _mlp_kernel,
        grid=grid,
        in_specs=[
            pl.BlockSpec((1, BLK, BOND_LEN), lambda b, j: (b, j, 0)),
            pl.BlockSpec((NC, BLK, BOND_LEN), lambda b, j: (0, j, 0)),
            pl.BlockSpec((1, BLK, SITE_LEN), lambda b, j: (b, j, 0)),
            pl.BlockSpec((B, STATE_LEN), lambda b, j: (0, 0)),
            pl.BlockSpec((BOND_LEN, H1), lambda b, j: (0, 0)),
            pl.BlockSpec((SITE_LEN, H1), lambda b, j: (0, 0)),
            pl.BlockSpec((STATE_LEN, H1), lambda b, j: (0, 0)),
            pl.BlockSpec((1, H1), lambda b, j: (0, 0)),
            pl.BlockSpec((H1, H2), lambda b, j: (0, 0)),
            pl.BlockSpec((1, H2), lambda b, j: (0, 0)),
            pl.BlockSpec((H2, SITE_LEN), lambda b, j: (0, 0)),
            pl.BlockSpec((1, SITE_LEN), lambda b, j: (0, 0)),
        ],
        out_specs=pl.BlockSpec((1, BLK, SITE_LEN), lambda b, j: (b, j, 0)),
        out_shape=jax.ShapeDtypeStruct((B, N_SITES, SITE_LEN), jnp.float32),
        compiler_params=pltpu.CompilerParams(
            dimension_semantics=("parallel", "parallel"),
        ),
    )(sums, counts, sites, states, w1p, w1s, w1t, b1, w2, b2, w3, b3)


def kernel(sites, bonds, states, indices1, W1, b1, W2, b2, W3, b3):
    sums, counts = _sc_scatter(bonds, indices1.reshape(NCHUNK, CH))
    w1p = W1[:BOND_LEN]
    w1s = W1[BOND_LEN:BOND_LEN + SITE_LEN]
    w1t = W1[BOND_LEN + SITE_LEN:]
    return _mlp(sums, counts, sites, states, w1p, w1s, w1t,
                b1.reshape(1, H1), W2, b2.reshape(1, H2),
                W3, b3.reshape(1, SITE_LEN))
